# baseline (speedup 1.0000x reference)
import functools

import jax
import jax.numpy as jnp
from jax import lax
from jax.experimental import pallas as pl
from jax.experimental.pallas import tpu as pltpu
from jax.experimental.pallas import tpu_sc as plsc

B = 16384
D = 64
W = 2 * D
L = 16
NC = 2
NS = 16
NW = NC * NS
RPW = B // NW
CH = 128
NCHUNK = RPW // CH


def _sc_body(e1_hbm, rel_hbm, e2_hbm, ent_hbm, rel_t_hbm,
             out_hbm,
             e1_v, rel_v, e2_v,
             a_t, r_t, b_t,
             s_v, out_v, sem):
    wid = lax.axis_index("s") * NC + lax.axis_index("c")
    row0 = wid * RPW

    def unpack2(ref, row, k, off):
        sl = pl.ds(off + k * 2 * L, 2 * L)
        return plsc.unpack(ref[row, sl], format=plsc.PackFormat.INTERLEAVED,
                           preferred_element_type=jnp.float32)

    def chunk_body(c, carry):
        base = row0 + c * CH
        pltpu.sync_copy(e1_hbm.at[pl.ds(base, CH)], e1_v)
        pltpu.sync_copy(rel_hbm.at[pl.ds(base, CH)], rel_v)
        pltpu.sync_copy(e2_hbm.at[pl.ds(base, CH)], e2_v)
        cps = [
            pltpu.async_copy(ent_hbm.at[e1_v], a_t, sem),
            pltpu.async_copy(rel_t_hbm.at[rel_v], r_t, sem),
            pltpu.async_copy(ent_hbm.at[e2_v], b_t, sem),
        ]
        for cp in cps:
            cp.wait()

        def group_body(g, carry2):
            def row_body(r, carry3):
                row = g * L + r
                acc = jnp.zeros((L,), jnp.float32)
                for k in range(2):
                    ar0, ar1 = unpack2(a_t, row, k, 0)
                    ai0, ai1 = unpack2(a_t, row, k, D)
                    rr0, rr1 = unpack2(r_t, row, k, 0)
                    ri0, ri1 = unpack2(r_t, row, k, D)
                    br0, br1 = unpack2(b_t, row, k, 0)
                    bi0, bi1 = unpack2(b_t, row, k, D)
                    acc = acc + br0 * (ar0 * rr0 - ai0 * ri0) + bi0 * (ar0 * ri0 + ai0 * rr0)
                    acc = acc + br1 * (ar1 * rr1 - ai1 * ri1) + bi1 * (ar1 * ri1 + ai1 * rr1)
                s_v[pl.ds(pl.multiple_of(r * L, L), L)] = acc
                return carry3

            lax.fori_loop(0, L, row_body, 0)
            lane = lax.iota(jnp.int32, L)
            tot = jnp.zeros((L,), jnp.float32)
            for j in range(L):
                tot = tot + plsc.load_gather(s_v, [lane * L + j])
            res = 1.0 / (1.0 + jnp.exp(-tot))
            off = pl.multiple_of(c * CH + g * L, L)
            out_v[pl.ds(off, L)] = res
            return carry2

        lax.fori_loop(0, CH // L, group_body, 0)
        return carry

    lax.fori_loop(0, NCHUNK, chunk_body, 0)
    pltpu.sync_copy(out_v, out_hbm.at[pl.ds(row0, RPW)])


@jax.jit
def _scores(e1_idx, rel_idx, e2_idx, ent_cat, rel_cat):
    mesh = plsc.VectorSubcoreMesh(core_axis_name="c", subcore_axis_name="s")
    fn = pl.kernel(
        _sc_body,
        mesh=mesh,
        compiler_params=pltpu.CompilerParams(
            needs_layout_passes=False, use_tc_tiling_on_sc=False
        ),
        out_type=jax.ShapeDtypeStruct((B,), jnp.float32),
        scratch_types=[
            pltpu.VMEM((CH,), jnp.int32),
            pltpu.VMEM((CH,), jnp.int32),
            pltpu.VMEM((CH,), jnp.int32),
            pltpu.VMEM((CH, W), jnp.bfloat16),
            pltpu.VMEM((CH, W), jnp.bfloat16),
            pltpu.VMEM((CH, W), jnp.bfloat16),
            pltpu.VMEM((L * L,), jnp.float32),
            pltpu.VMEM((RPW,), jnp.float32),
            pltpu.SemaphoreType.DMA,
        ],
    )
    return fn(e1_idx, rel_idx, e2_idx, ent_cat, rel_cat)


def kernel(e1_idx, rel_idx, e2_idx, ent_real, ent_img, rel_real, rel_img):
    e1 = e1_idx.astype(jnp.int32)
    rel = rel_idx.astype(jnp.int32)
    e2 = e2_idx.astype(jnp.int32)
    ent_cat = jnp.concatenate(
        [ent_real, ent_img], axis=1).astype(jnp.bfloat16)
    rel_cat = jnp.concatenate(
        [rel_real, rel_img], axis=1).astype(jnp.bfloat16)
    out = _scores(e1, rel, e2, ent_cat, rel_cat)
    return (out, jnp.float32(0.0))


# speedup vs baseline: 1.0023x; 1.0023x over previous
import functools

import jax
import jax.numpy as jnp
from jax import lax
from jax.experimental import pallas as pl
from jax.experimental.pallas import tpu as pltpu
from jax.experimental.pallas import tpu_sc as plsc

B = 16384
D = 64
L = 16
NC = 2
NS = 16
NW = NC * NS
RPW = B // NW
CH = 128
NCHUNK = RPW // CH


def _sc_body(e1_hbm, rel_hbm, e2_hbm, er_hbm, ei_hbm, rr_hbm, ri_hbm,
             out_hbm,
             e1_v, rel_v, e2_v,
             a_r, a_i, r_r, r_i, b_r, b_i,
             s_v, out_v, sem):
    wid = lax.axis_index("s") * NC + lax.axis_index("c")
    row0 = wid * RPW

    def unpack2(ref, row, k):
        sl = pl.ds(k * 2 * L, 2 * L)
        return plsc.unpack(ref[row, sl], format=plsc.PackFormat.INTERLEAVED,
                           preferred_element_type=jnp.float32)

    def chunk_body(c, carry):
        base = row0 + c * CH
        pltpu.sync_copy(e1_hbm.at[pl.ds(base, CH)], e1_v)
        pltpu.sync_copy(rel_hbm.at[pl.ds(base, CH)], rel_v)
        pltpu.sync_copy(e2_hbm.at[pl.ds(base, CH)], e2_v)
        cps = [
            pltpu.async_copy(er_hbm.at[e1_v], a_r, sem),
            pltpu.async_copy(ei_hbm.at[e1_v], a_i, sem),
            pltpu.async_copy(rr_hbm.at[rel_v], r_r, sem),
            pltpu.async_copy(ri_hbm.at[rel_v], r_i, sem),
            pltpu.async_copy(er_hbm.at[e2_v], b_r, sem),
            pltpu.async_copy(ei_hbm.at[e2_v], b_i, sem),
        ]
        for cp in cps:
            cp.wait()

        def group_body(g, carry2):
            def row_body(r, carry3):
                row = g * L + r
                acc = jnp.zeros((L,), jnp.float32)
                for k in range(2):
                    ar0, ar1 = unpack2(a_r, row, k)
                    ai0, ai1 = unpack2(a_i, row, k)
                    rr0, rr1 = unpack2(r_r, row, k)
                    ri0, ri1 = unpack2(r_i, row, k)
                    br0, br1 = unpack2(b_r, row, k)
                    bi0, bi1 = unpack2(b_i, row, k)
                    acc = acc + br0 * (ar0 * rr0 - ai0 * ri0) + bi0 * (ar0 * ri0 + ai0 * rr0)
                    acc = acc + br1 * (ar1 * rr1 - ai1 * ri1) + bi1 * (ar1 * ri1 + ai1 * rr1)
                s_v[pl.ds(pl.multiple_of(r * L, L), L)] = acc
                return carry3

            lax.fori_loop(0, L, row_body, 0)
            lane = lax.iota(jnp.int32, L)
            tot = jnp.zeros((L,), jnp.float32)
            for j in range(L):
                tot = tot + plsc.load_gather(s_v, [lane * L + j])
            res = 1.0 / (1.0 + jnp.exp(-tot))
            off = pl.multiple_of(c * CH + g * L, L)
            out_v[pl.ds(off, L)] = res
            return carry2

        lax.fori_loop(0, CH // L, group_body, 0)
        return carry

    lax.fori_loop(0, NCHUNK, chunk_body, 0)
    pltpu.sync_copy(out_v, out_hbm.at[pl.ds(row0, RPW)])


@jax.jit
def _scores(e1_idx, rel_idx, e2_idx, ent_real, ent_img, rel_real, rel_img):
    mesh = plsc.VectorSubcoreMesh(core_axis_name="c", subcore_axis_name="s")
    fn = pl.kernel(
        _sc_body,
        mesh=mesh,
        compiler_params=pltpu.CompilerParams(
            needs_layout_passes=False, use_tc_tiling_on_sc=False
        ),
        out_type=jax.ShapeDtypeStruct((B,), jnp.float32),
        scratch_types=[
            pltpu.VMEM((CH,), jnp.int32),
            pltpu.VMEM((CH,), jnp.int32),
            pltpu.VMEM((CH,), jnp.int32),
            pltpu.VMEM((CH, D), jnp.bfloat16),
            pltpu.VMEM((CH, D), jnp.bfloat16),
            pltpu.VMEM((CH, D), jnp.bfloat16),
            pltpu.VMEM((CH, D), jnp.bfloat16),
            pltpu.VMEM((CH, D), jnp.bfloat16),
            pltpu.VMEM((CH, D), jnp.bfloat16),
            pltpu.VMEM((L * L,), jnp.float32),
            pltpu.VMEM((RPW,), jnp.float32),
            pltpu.SemaphoreType.DMA,
        ],
    )
    return fn(e1_idx, rel_idx, e2_idx, ent_real, ent_img, rel_real, rel_img)


def kernel(e1_idx, rel_idx, e2_idx, ent_real, ent_img, rel_real, rel_img):
    e1 = e1_idx.astype(jnp.int32)
    rel = rel_idx.astype(jnp.int32)
    e2 = e2_idx.astype(jnp.int32)
    er, ei, rr, ri = lax.optimization_barrier(
        (ent_real.astype(jnp.bfloat16), ent_img.astype(jnp.bfloat16),
         rel_real.astype(jnp.bfloat16), rel_img.astype(jnp.bfloat16)))
    out = _scores(e1, rel, e2, er, ei, rr, ri)
    return (out, jnp.float32(0.0))


# speedup vs baseline: 1.3161x; 1.3131x over previous
import functools

import jax
import jax.numpy as jnp
from jax import lax
from jax.experimental import pallas as pl
from jax.experimental.pallas import tpu as pltpu
from jax.experimental.pallas import tpu_sc as plsc

B = 16384
D = 64
W = 128
L = 16
NC = 2
NS = 16
NW = NC * NS
RPW = B // NW
CH = 128
NCHUNK = RPW // CH


def _sc_body(e1_hbm, rel_hbm, e2_hbm, er_hbm, ei_hbm, rr_hbm, ri_hbm,
             out_hbm,
             e1_v, rel_v, e2_v, e1w_v, relw_v, e2w_v,
             pa_v, pr_v, pb_v,
             a_r, a_i, r_r, r_i, b_r, b_i,
             out_v, sem):
    wid = lax.axis_index("s") * NC + lax.axis_index("c")
    row0 = wid * RPW

    def chunk_body(c, carry):
        base = row0 + c * CH
        pltpu.sync_copy(e1_hbm.at[pl.ds(base, CH)], e1_v)
        pltpu.sync_copy(rel_hbm.at[pl.ds(base, CH)], rel_v)
        pltpu.sync_copy(e2_hbm.at[pl.ds(base, CH)], e2_v)

        def halve_body(i, carry2):
            sl = pl.ds(pl.multiple_of(i * L, L), L)
            e1c = e1_v[sl]
            rlc = rel_v[sl]
            e2c = e2_v[sl]
            e1w_v[sl] = lax.shift_right_logical(e1c, 1)
            relw_v[sl] = lax.shift_right_logical(rlc, 1)
            e2w_v[sl] = lax.shift_right_logical(e2c, 1)
            pa_v[sl] = (e1c & 1) * D
            pr_v[sl] = (rlc & 1) * D
            pb_v[sl] = (e2c & 1) * D
            return carry2

        lax.fori_loop(0, CH // L, halve_body, 0)

        cps = [
            pltpu.async_copy(er_hbm.at[e1w_v], a_r, sem),
            pltpu.async_copy(ei_hbm.at[e1w_v], a_i, sem),
            pltpu.async_copy(rr_hbm.at[relw_v], r_r, sem),
            pltpu.async_copy(ri_hbm.at[relw_v], r_i, sem),
            pltpu.async_copy(er_hbm.at[e2w_v], b_r, sem),
            pltpu.async_copy(ei_hbm.at[e2w_v], b_i, sem),
        ]
        for cp in cps:
            cp.wait()

        def group_body(g, carry2):
            sl = pl.ds(pl.multiple_of(g * L, L), L)
            rowv = g * L + lax.iota(jnp.int32, L)
            pav = pa_v[sl]
            prv = pr_v[sl]
            pbv = pb_v[sl]
            acc = jnp.zeros((L,), jnp.float32)
            for d in range(D):
                ca = pav + d
                cr = prv + d
                cb = pbv + d
                ar = plsc.load_gather(a_r, [rowv, ca])
                ai = plsc.load_gather(a_i, [rowv, ca])
                rr = plsc.load_gather(r_r, [rowv, cr])
                ri = plsc.load_gather(r_i, [rowv, cr])
                br = plsc.load_gather(b_r, [rowv, cb])
                bi = plsc.load_gather(b_i, [rowv, cb])
                acc = acc + br * (ar * rr - ai * ri) + bi * (ar * ri + ai * rr)
            res = 1.0 / (1.0 + jnp.exp(-acc))
            off = pl.multiple_of(c * CH + g * L, L)
            out_v[pl.ds(off, L)] = res
            return carry2

        lax.fori_loop(0, CH // L, group_body, 0)
        return carry

    lax.fori_loop(0, NCHUNK, chunk_body, 0)
    pltpu.sync_copy(out_v, out_hbm.at[pl.ds(row0, RPW)])


@jax.jit
def _scores(e1_idx, rel_idx, e2_idx, ent_real2, ent_img2, rel_real2, rel_img2):
    mesh = plsc.VectorSubcoreMesh(core_axis_name="c", subcore_axis_name="s")
    fn = pl.kernel(
        _sc_body,
        mesh=mesh,
        compiler_params=pltpu.CompilerParams(needs_layout_passes=False),
        out_type=jax.ShapeDtypeStruct((B,), jnp.float32),
        scratch_types=[
            pltpu.VMEM((CH,), jnp.int32),
            pltpu.VMEM((CH,), jnp.int32),
            pltpu.VMEM((CH,), jnp.int32),
            pltpu.VMEM((CH,), jnp.int32),
            pltpu.VMEM((CH,), jnp.int32),
            pltpu.VMEM((CH,), jnp.int32),
            pltpu.VMEM((CH,), jnp.int32),
            pltpu.VMEM((CH,), jnp.int32),
            pltpu.VMEM((CH,), jnp.int32),
            pltpu.VMEM((CH, W), jnp.float32),
            pltpu.VMEM((CH, W), jnp.float32),
            pltpu.VMEM((CH, W), jnp.float32),
            pltpu.VMEM((CH, W), jnp.float32),
            pltpu.VMEM((CH, W), jnp.float32),
            pltpu.VMEM((CH, W), jnp.float32),
            pltpu.VMEM((RPW,), jnp.float32),
            pltpu.SemaphoreType.DMA,
        ],
    )
    return fn(e1_idx, rel_idx, e2_idx, ent_real2, ent_img2, rel_real2, rel_img2)


def kernel(e1_idx, rel_idx, e2_idx, ent_real, ent_img, rel_real, rel_img):
    e1 = e1_idx.astype(jnp.int32)
    rel = rel_idx.astype(jnp.int32)
    e2 = e2_idx.astype(jnp.int32)
    ne, d = ent_real.shape
    nr = rel_real.shape[0]
    er2 = ent_real.reshape(ne // 2, 2 * d)
    ei2 = ent_img.reshape(ne // 2, 2 * d)
    rr2 = rel_real.reshape(nr // 2, 2 * d)
    ri2 = rel_img.reshape(nr // 2, 2 * d)
    out = _scores(e1, rel, e2, er2, ei2, rr2, ri2)
    return (out, jnp.float32(0.0))


# speedup vs baseline: 1.4495x; 1.1014x over previous
import functools

import jax
import jax.numpy as jnp
from jax import lax
from jax.experimental import pallas as pl
from jax.experimental.pallas import tpu as pltpu
from jax.experimental.pallas import tpu_sc as plsc

B = 16384
D = 64
L = 16
NC = 2
NS = 16
NW = NC * NS
RPW = B // NW
CH = 128
NCHUNK = RPW // CH


def _sc_body(e1_hbm, rel_hbm, e2_hbm, er_hbm, ei_hbm, rr_hbm, ri_hbm,
             out_hbm,
             e1_v, rel_v, e2_v,
             a_r0, a_i0, r_r0, r_i0, b_r0, b_i0,
             a_r1, a_i1, r_r1, r_i1, b_r1, b_i1,
             s_v, out_v, sem0, sem1):
    wid = lax.axis_index("s") * NC + lax.axis_index("c")
    row0 = wid * RPW

    pltpu.sync_copy(e1_hbm.at[pl.ds(row0, RPW)], e1_v)
    pltpu.sync_copy(rel_hbm.at[pl.ds(row0, RPW)], rel_v)
    pltpu.sync_copy(e2_hbm.at[pl.ds(row0, RPW)], e2_v)

    bufs = [
        (a_r0, a_i0, r_r0, r_i0, b_r0, b_i0, sem0),
        (a_r1, a_i1, r_r1, r_i1, b_r1, b_i1, sem1),
    ]

    def start(c, buf):
        a_r, a_i, r_r, r_i, b_r, b_i, sem = buf
        sl = pl.ds(c * CH, CH)
        return [
            pltpu.async_copy(er_hbm.at[e1_v.at[sl]], a_r, sem),
            pltpu.async_copy(ei_hbm.at[e1_v.at[sl]], a_i, sem),
            pltpu.async_copy(rr_hbm.at[rel_v.at[sl]], r_r, sem),
            pltpu.async_copy(ri_hbm.at[rel_v.at[sl]], r_i, sem),
            pltpu.async_copy(er_hbm.at[e2_v.at[sl]], b_r, sem),
            pltpu.async_copy(ei_hbm.at[e2_v.at[sl]], b_i, sem),
        ]

    def compute(c, buf):
        a_r, a_i, r_r, r_i, b_r, b_i, _ = buf

        def group_body(g, carry2):
            def row_body(r, carry3):
                row = g * L + r
                acc = jnp.zeros((L,), jnp.float32)
                for k in range(D // L):
                    sl = pl.ds(k * L, L)
                    ar = a_r[row, sl]
                    ai = a_i[row, sl]
                    rr = r_r[row, sl]
                    ri = r_i[row, sl]
                    br = b_r[row, sl]
                    bi = b_i[row, sl]
                    acc = acc + br * (ar * rr - ai * ri) + bi * (ar * ri + ai * rr)
                s_v[pl.ds(pl.multiple_of(r * L, L), L)] = acc
                return carry3

            lax.fori_loop(0, L, row_body, 0)
            lane = lax.iota(jnp.int32, L)
            tot = jnp.zeros((L,), jnp.float32)
            for j in range(L):
                tot = tot + plsc.load_gather(s_v, [lane * L + j])
            res = 1.0 / (1.0 + jnp.exp(-tot))
            off = pl.multiple_of(c * CH + g * L, L)
            out_v[pl.ds(off, L)] = res
            return carry2

        lax.fori_loop(0, CH // L, group_body, 0)

    cps = start(0, bufs[0])
    for c in range(NCHUNK):
        nxt = None
        if c + 1 < NCHUNK:
            nxt = start(c + 1, bufs[(c + 1) % 2])
        for cp in cps:
            cp.wait()
        compute(c, bufs[c % 2])
        cps = nxt

    pltpu.sync_copy(out_v, out_hbm.at[pl.ds(row0, RPW)])


@jax.jit
def _scores(e1_idx, rel_idx, e2_idx, ent_real, ent_img, rel_real, rel_img):
    mesh = plsc.VectorSubcoreMesh(core_axis_name="c", subcore_axis_name="s")
    fn = pl.kernel(
        _sc_body,
        mesh=mesh,
        compiler_params=pltpu.CompilerParams(
            needs_layout_passes=False, use_tc_tiling_on_sc=False
        ),
        out_type=jax.ShapeDtypeStruct((B,), jnp.float32),
        scratch_types=(
            [pltpu.VMEM((RPW,), jnp.int32)] * 3
            + [pltpu.VMEM((CH, D), jnp.float32)] * 12
            + [
                pltpu.VMEM((L * L,), jnp.float32),
                pltpu.VMEM((RPW,), jnp.float32),
                pltpu.SemaphoreType.DMA,
                pltpu.SemaphoreType.DMA,
            ]
        ),
    )
    return fn(e1_idx, rel_idx, e2_idx, ent_real, ent_img, rel_real, rel_img)


def kernel(e1_idx, rel_idx, e2_idx, ent_real, ent_img, rel_real, rel_img):
    e1 = e1_idx.astype(jnp.int32)
    rel = rel_idx.astype(jnp.int32)
    e2 = e2_idx.astype(jnp.int32)
    out = _scores(e1, rel, e2, ent_real, ent_img, rel_real, rel_img)
    return (out, jnp.float32(0.0))
